# initial kernel scaffold (unmeasured)
import jax
import jax.numpy as jnp
from jax import lax
from jax.experimental import pallas as pl
from jax.experimental.pallas import tpu as pltpu

N_DEV = 32
SQ = 1024
HQ = 8
DH = 128
D = HQ * DH
SKV_LOCAL = 1024
R = 4
QG = SQ // R
KG = SKV_LOCAL // R
CHUNK = SQ // N_DEV
EXT = D + 128
SCALE = 0.08838834764831843


def _orig_start(c):
    r = c // 8
    a = (c % 8) // 2
    t0 = (c % 2) * 32
    return (4 * a + r) * 64 + t0


def kernel(x, Wq, K_ext, V_ext, Wo):
    x2 = x.reshape(SQ, D)
    k2 = K_ext.reshape(SKV_LOCAL, HQ, DH)
    v2 = V_ext.reshape(SKV_LOCAL, HQ, DH)

    def body(x_ref, wq_ref, k_ref, v_ref, wo_ref, out_ref,
             qbf_ref, qp_ref, kp_ref, vp_ref, wobf_ref,
             oext_ref, ctx_ref, rs_comm, ag_comm,
             rs_send_sems, rs_recv_sems, ag_send_sems, ag_recv_sems):
        my = lax.axis_index("i")
        left = (my - 1) % N_DEV
        right = (my + 1) % N_DEV

        barrier_sem = pltpu.get_barrier_semaphore()
        for nbr in (left, right):
            pl.semaphore_signal(
                barrier_sem, inc=1,
                device_id=(nbr,), device_id_type=pl.DeviceIdType.MESH,
            )
        pl.semaphore_wait(barrier_sem, 2)

        wobf_ref[...] = wo_ref[...].astype(jnp.bfloat16)
        qbf_ref[...] = jnp.dot(
            x_ref[...].astype(jnp.bfloat16),
            wq_ref[...].astype(jnp.bfloat16),
            preferred_element_type=jnp.bfloat16,
        )
        for t in range(16):
            r, a = t % 4, t // 4
            dst = r * 256 + a * 64
            qp_ref[dst:dst + 64, :] = qbf_ref[t * 64:(t + 1) * 64, :]
            kp_ref[dst:dst + 64, :] = (
                k_ref[t * 64:(t + 1) * 64, :, :].reshape(64, D)
                .astype(jnp.bfloat16))
            vp_ref[dst:dst + 64, :] = (
                v_ref[t * 64:(t + 1) * 64, :, :].reshape(64, D)
                .astype(jnp.bfloat16))

        oext_ref[:, D:] = jnp.zeros((SQ, 128), jnp.float32)
        for r in range(R):
            rows = slice(r * QG, (r + 1) * QG)
            for h in range(HQ):
                cols = slice(h * DH, (h + 1) * DH)
                qh = qp_ref[rows, cols]
                kh = kp_ref[rows, cols]
                s = lax.dot_general(
                    qh, kh, (((1,), (1,)), ((), ())),
                    preferred_element_type=jnp.float32)
                w = jnp.exp(s * SCALE)
                o = jnp.dot(w.astype(jnp.bfloat16), vp_ref[rows, cols],
                            preferred_element_type=jnp.float32)
                oext_ref[rows, cols] = o
                oext_ref[rows, D + h:D + h + 1] = jnp.sum(
                    w, axis=1, keepdims=True)

        rs_comm[0, :, :] = oext_ref[pl.ds(my * CHUNK, CHUNK), :]
        for s in range(N_DEV - 1):
            send_slot = s % 2
            recv_slot = (s + 1) % 2
            rdma = pltpu.make_async_remote_copy(
                src_ref=rs_comm.at[send_slot],
                dst_ref=rs_comm.at[recv_slot],
                send_sem=rs_send_sems.at[send_slot],
                recv_sem=rs_recv_sems.at[recv_slot],
                device_id=(right,),
                device_id_type=pl.DeviceIdType.MESH,
            )
            rdma.start()
            rdma.wait()
            c = (my - s - 1) % N_DEV
            rs_comm[recv_slot, :, :] = (
                rs_comm[recv_slot, :, :]
                + oext_ref[pl.ds(c * CHUNK, CHUNK), :])

        final_slot = (N_DEV - 1) % 2
        for h in range(HQ):
            oh = rs_comm[final_slot, :, h * DH:(h + 1) * DH]
            lh = rs_comm[final_slot, :, D + h:D + h + 1]
            ctx_ref[:, h * DH:(h + 1) * DH] = (oh / lh).astype(jnp.bfloat16)
        out_chunk = jnp.dot(ctx_ref[...], wobf_ref[...],
                            preferred_element_type=jnp.float32)

        own_c = (my + 1) % N_DEV
        out_ref[pl.ds(_orig_start(own_c), CHUNK), :] = out_chunk
        ag_comm[0, :, :] = out_chunk

        for s in range(N_DEV - 1):
            send_slot = s % 2
            recv_slot = (s + 1) % 2
            rdma = pltpu.make_async_remote_copy(
                src_ref=ag_comm.at[send_slot],
                dst_ref=ag_comm.at[recv_slot],
                send_sem=ag_send_sems.at[send_slot],
                recv_sem=ag_recv_sems.at[recv_slot],
                device_id=(right,),
                device_id_type=pl.DeviceIdType.MESH,
            )
            rdma.start()
            rdma.wait()
            c = (my - s) % N_DEV
            out_ref[pl.ds(_orig_start(c), CHUNK), :] = ag_comm[recv_slot, :, :]

    out = pl.pallas_call(
        body,
        out_shape=jax.ShapeDtypeStruct((SQ, D), jnp.float32),
        in_specs=[pl.BlockSpec(memory_space=pltpu.VMEM)] * 5,
        out_specs=pl.BlockSpec(memory_space=pltpu.VMEM),
        scratch_shapes=[
            pltpu.VMEM((SQ, D), jnp.bfloat16),
            pltpu.VMEM((SQ, D), jnp.bfloat16),
            pltpu.VMEM((SKV_LOCAL, D), jnp.bfloat16),
            pltpu.VMEM((SKV_LOCAL, D), jnp.bfloat16),
            pltpu.VMEM((D, D), jnp.bfloat16),
            pltpu.VMEM((SQ, EXT), jnp.float32),
            pltpu.VMEM((CHUNK, D), jnp.bfloat16),
            pltpu.VMEM((2, CHUNK, EXT), jnp.float32),
            pltpu.VMEM((2, CHUNK, D), jnp.float32),
            pltpu.SemaphoreType.DMA((2,)),
            pltpu.SemaphoreType.DMA((2,)),
            pltpu.SemaphoreType.DMA((2,)),
            pltpu.SemaphoreType.DMA((2,)),
        ],
        compiler_params=pltpu.CompilerParams(collective_id=0),
    )(x2, Wq, k2, v2, Wo)
    return out.reshape(1, SQ, D)


# baseline (device time: 232106 ns/iter reference)
import jax
import jax.numpy as jnp
from jax import lax
from jax.experimental import pallas as pl
from jax.experimental.pallas import tpu as pltpu

N_DEV = 32
SQ = 1024
HQ = 8
DH = 128
D = HQ * DH
SKV_LOCAL = 1024
R = 4
QG = SQ // R
KG = SKV_LOCAL // R
CHUNK = SQ // N_DEV
EXT = D + 128
SCALE = 0.08838834764831843


def _orig_start(c):
    r = c // 8
    a = (c % 8) // 2
    t0 = (c % 2) * 32
    return (4 * a + r) * 64 + t0


def kernel(x, Wq, K_ext, V_ext, Wo):
    x2 = x.reshape(SQ, D)
    k2 = K_ext.reshape(SKV_LOCAL, HQ, DH)
    v2 = V_ext.reshape(SKV_LOCAL, HQ, DH)

    def body(x_ref, wq_ref, k_ref, v_ref, wo_ref, out_ref,
             qbf_ref, qp_ref, kp_ref, vp_ref, wobf_ref,
             oext_ref, ctx_ref, rs_comm, ag_comm,
             rs_send_sems, rs_recv_sems, ag_send_sems, ag_recv_sems):
        my = lax.axis_index("i")
        left = (my - 1) % N_DEV
        right = (my + 1) % N_DEV

        barrier_sem = pltpu.get_barrier_semaphore()
        for nbr in (left, right):
            pl.semaphore_signal(
                barrier_sem, inc=1,
                device_id=(nbr,), device_id_type=pl.DeviceIdType.MESH,
            )
        pl.semaphore_wait(barrier_sem, 2)

        wobf_ref[...] = wo_ref[...].astype(jnp.bfloat16)
        qbf_ref[...] = jnp.dot(
            x_ref[...].astype(jnp.bfloat16),
            wq_ref[...].astype(jnp.bfloat16),
            preferred_element_type=jnp.float32,
        ).astype(jnp.bfloat16)
        for t in range(16):
            r, a = t % 4, t // 4
            dst = r * 256 + a * 64
            qp_ref[dst:dst + 64, :] = qbf_ref[t * 64:(t + 1) * 64, :]
            kp_ref[dst:dst + 64, :] = (
                k_ref[t * 64:(t + 1) * 64, :, :].reshape(64, D)
                .astype(jnp.bfloat16))
            vp_ref[dst:dst + 64, :] = (
                v_ref[t * 64:(t + 1) * 64, :, :].reshape(64, D)
                .astype(jnp.bfloat16))

        oext_ref[:, D:] = jnp.zeros((SQ, 128), jnp.float32)
        for r in range(R):
            rows = slice(r * QG, (r + 1) * QG)
            for h in range(HQ):
                cols = slice(h * DH, (h + 1) * DH)
                qh = qp_ref[rows, cols]
                kh = kp_ref[rows, cols]
                s = lax.dot_general(
                    qh, kh, (((1,), (1,)), ((), ())),
                    preferred_element_type=jnp.float32)
                w = jnp.exp(s * SCALE)
                o = jnp.dot(w.astype(jnp.bfloat16), vp_ref[rows, cols],
                            preferred_element_type=jnp.float32)
                oext_ref[rows, cols] = o
                oext_ref[rows, D + h:D + h + 1] = jnp.sum(
                    w, axis=1, keepdims=True)

        rs_comm[0, :, :] = oext_ref[pl.ds(my * CHUNK, CHUNK), :]
        for s in range(N_DEV - 1):
            send_slot = s % 2
            recv_slot = (s + 1) % 2
            rdma = pltpu.make_async_remote_copy(
                src_ref=rs_comm.at[send_slot],
                dst_ref=rs_comm.at[recv_slot],
                send_sem=rs_send_sems.at[send_slot],
                recv_sem=rs_recv_sems.at[recv_slot],
                device_id=(right,),
                device_id_type=pl.DeviceIdType.MESH,
            )
            rdma.start()
            rdma.wait()
            c = (my - s - 1) % N_DEV
            rs_comm[recv_slot, :, :] = (
                rs_comm[recv_slot, :, :]
                + oext_ref[pl.ds(c * CHUNK, CHUNK), :])

        final_slot = (N_DEV - 1) % 2
        for h in range(HQ):
            oh = rs_comm[final_slot, :, h * DH:(h + 1) * DH]
            lh = rs_comm[final_slot, :, D + h:D + h + 1]
            ctx_ref[:, h * DH:(h + 1) * DH] = (oh / lh).astype(jnp.bfloat16)
        out_chunk = jnp.dot(ctx_ref[...], wobf_ref[...],
                            preferred_element_type=jnp.float32)

        own_c = (my + 1) % N_DEV
        out_ref[pl.ds(_orig_start(own_c), CHUNK), :] = out_chunk
        ag_comm[0, :, :] = out_chunk

        for s in range(N_DEV - 1):
            send_slot = s % 2
            recv_slot = (s + 1) % 2
            rdma = pltpu.make_async_remote_copy(
                src_ref=ag_comm.at[send_slot],
                dst_ref=ag_comm.at[recv_slot],
                send_sem=ag_send_sems.at[send_slot],
                recv_sem=ag_recv_sems.at[recv_slot],
                device_id=(right,),
                device_id_type=pl.DeviceIdType.MESH,
            )
            rdma.start()
            rdma.wait()
            c = (my - s) % N_DEV
            out_ref[pl.ds(_orig_start(c), CHUNK), :] = ag_comm[recv_slot, :, :]

    out = pl.pallas_call(
        body,
        out_shape=jax.ShapeDtypeStruct((SQ, D), jnp.float32),
        in_specs=[pl.BlockSpec(memory_space=pltpu.VMEM)] * 5,
        out_specs=pl.BlockSpec(memory_space=pltpu.VMEM),
        scratch_shapes=[
            pltpu.VMEM((SQ, D), jnp.bfloat16),
            pltpu.VMEM((SQ, D), jnp.bfloat16),
            pltpu.VMEM((SKV_LOCAL, D), jnp.bfloat16),
            pltpu.VMEM((SKV_LOCAL, D), jnp.bfloat16),
            pltpu.VMEM((D, D), jnp.bfloat16),
            pltpu.VMEM((SQ, EXT), jnp.float32),
            pltpu.VMEM((CHUNK, D), jnp.bfloat16),
            pltpu.VMEM((2, CHUNK, EXT), jnp.float32),
            pltpu.VMEM((2, CHUNK, D), jnp.float32),
            pltpu.SemaphoreType.DMA((2,)),
            pltpu.SemaphoreType.DMA((2,)),
            pltpu.SemaphoreType.DMA((2,)),
            pltpu.SemaphoreType.DMA((2,)),
        ],
        compiler_params=pltpu.CompilerParams(collective_id=0),
    )(x2, Wq, k2, v2, Wo)
    return out.reshape(1, SQ, D)


# device time: 102074 ns/iter; 2.2739x vs baseline; 2.2739x over previous
import jax
import jax.numpy as jnp
from jax import lax
from jax.experimental import pallas as pl
from jax.experimental.pallas import tpu as pltpu

N_DEV = 32
SQ = 1024
HQ = 8
DH = 128
D = HQ * DH
SKV_LOCAL = 1024
R = 4
QG = SQ // R
KG = SKV_LOCAL // R
CHUNK = SQ // N_DEV
EXT = D + 128
SCALE = 0.08838834764831843

RS_M = (8, 16, 1, 2, 4)
RS_S = (16, 8, 4, 2, 1)
RS_OFF = (0, 512, 768, 896, 960)
AG_M = (4, 2, 1, 16, 8)


def _orig_start(c):
    r = c // 8
    a = (c % 8) // 2
    t0 = (c % 2) * 32
    return (4 * a + r) * 64 + t0


def kernel(x, Wq, K_ext, V_ext, Wo):
    x2 = x.reshape(SQ, D)
    k2 = K_ext.reshape(SKV_LOCAL, HQ, DH)
    v2 = V_ext.reshape(SKV_LOCAL, HQ, DH)

    def body(x_ref, wq_ref, k_ref, v_ref, wo_ref, out_ref,
             qbf_ref, qp_ref, kp_ref, vp_ref, wobf_ref,
             oext_ref, ctx_ref, send_buf, recv_buf, out_perm,
             rs_send_sems, rs_recv_sems, ag_send_sems, ag_recv_sems):
        my = lax.axis_index("i")

        barrier_sem = pltpu.get_barrier_semaphore()
        for m in RS_M:
            pl.semaphore_signal(
                barrier_sem, inc=1,
                device_id=(my ^ m,), device_id_type=pl.DeviceIdType.MESH,
            )
        pl.semaphore_wait(barrier_sem, len(RS_M))

        wobf_ref[...] = wo_ref[...].astype(jnp.bfloat16)
        qbf_ref[...] = jnp.dot(
            x_ref[...].astype(jnp.bfloat16),
            wq_ref[...].astype(jnp.bfloat16),
            preferred_element_type=jnp.float32,
        ).astype(jnp.bfloat16)
        for t in range(16):
            r, a = t % 4, t // 4
            dst = r * 256 + a * 64
            qp_ref[dst:dst + 64, :] = qbf_ref[t * 64:(t + 1) * 64, :]
            kp_ref[dst:dst + 64, :] = (
                k_ref[t * 64:(t + 1) * 64, :, :].reshape(64, D)
                .astype(jnp.bfloat16))
            vp_ref[dst:dst + 64, :] = (
                v_ref[t * 64:(t + 1) * 64, :, :].reshape(64, D)
                .astype(jnp.bfloat16))

        oext_ref[:, D:] = jnp.zeros((SQ, 128), jnp.float32)
        for r in range(R):
            rows = slice(r * QG, (r + 1) * QG)
            for h in range(HQ):
                cols = slice(h * DH, (h + 1) * DH)
                qh = qp_ref[rows, cols]
                kh = kp_ref[rows, cols]
                s = lax.dot_general(
                    qh, kh, (((1,), (1,)), ((), ())),
                    preferred_element_type=jnp.float32)
                w = jnp.exp(s * SCALE)
                o = jnp.dot(w.astype(jnp.bfloat16), vp_ref[rows, cols],
                            preferred_element_type=jnp.float32)
                oext_ref[rows, cols] = o
                oext_ref[rows, D + h:D + h + 1] = jnp.sum(
                    w, axis=1, keepdims=True)

        lo = my * 0
        for k in range(5):
            m, s_ch, off = RS_M[k], RS_S[k], RS_OFF[k]
            rows = s_ch * CHUNK
            bit = (my // m) % 2
            keep_lo = lo + bit * s_ch
            send_lo = lo + (1 - bit) * s_ch
            send_buf[0:rows, :] = (
                oext_ref[pl.ds(send_lo * CHUNK, rows), :]
                .astype(jnp.bfloat16))
            rdma = pltpu.make_async_remote_copy(
                src_ref=send_buf.at[pl.ds(0, rows)],
                dst_ref=recv_buf.at[pl.ds(off, rows)],
                send_sem=rs_send_sems.at[k],
                recv_sem=rs_recv_sems.at[k],
                device_id=(my ^ m,),
                device_id_type=pl.DeviceIdType.MESH,
            )
            rdma.start()
            rdma.wait()
            oext_ref[pl.ds(keep_lo * CHUNK, rows), :] = (
                oext_ref[pl.ds(keep_lo * CHUNK, rows), :]
                + recv_buf[off:off + rows, :].astype(jnp.float32))
            lo = keep_lo

        for h in range(HQ):
            oh = oext_ref[pl.ds(lo * CHUNK, CHUNK), h * DH:(h + 1) * DH]
            lh = oext_ref[pl.ds(lo * CHUNK, CHUNK), D + h:D + h + 1]
            ctx_ref[:, h * DH:(h + 1) * DH] = (oh / lh).astype(jnp.bfloat16)
        out_chunk = jnp.dot(ctx_ref[...], wobf_ref[...],
                            preferred_element_type=jnp.float32)
        out_perm[pl.ds(lo * CHUNK, CHUNK), :] = out_chunk.astype(jnp.bfloat16)

        glo, gs = lo, 1
        for j in range(5):
            rows = gs * CHUNK
            rdma = pltpu.make_async_remote_copy(
                src_ref=out_perm.at[pl.ds(glo * CHUNK, rows)],
                dst_ref=out_perm.at[pl.ds(glo * CHUNK, rows)],
                send_sem=ag_send_sems.at[j],
                recv_sem=ag_recv_sems.at[j],
                device_id=(my ^ AG_M[j],),
                device_id_type=pl.DeviceIdType.MESH,
            )
            rdma.start()
            rdma.wait()
            glo = (glo // (2 * gs)) * (2 * gs)
            gs *= 2

        for c in range(N_DEV):
            st = _orig_start(c)
            out_ref[st:st + CHUNK, :] = (
                out_perm[c * CHUNK:(c + 1) * CHUNK, :].astype(jnp.float32))

    out = pl.pallas_call(
        body,
        out_shape=jax.ShapeDtypeStruct((SQ, D), jnp.float32),
        in_specs=[pl.BlockSpec(memory_space=pltpu.VMEM)] * 5,
        out_specs=pl.BlockSpec(memory_space=pltpu.VMEM),
        scratch_shapes=[
            pltpu.VMEM((SQ, D), jnp.bfloat16),
            pltpu.VMEM((SQ, D), jnp.bfloat16),
            pltpu.VMEM((SKV_LOCAL, D), jnp.bfloat16),
            pltpu.VMEM((SKV_LOCAL, D), jnp.bfloat16),
            pltpu.VMEM((D, D), jnp.bfloat16),
            pltpu.VMEM((SQ, EXT), jnp.float32),
            pltpu.VMEM((CHUNK, D), jnp.bfloat16),
            pltpu.VMEM((512, EXT), jnp.bfloat16),
            pltpu.VMEM((992, EXT), jnp.bfloat16),
            pltpu.VMEM((SQ, D), jnp.bfloat16),
            pltpu.SemaphoreType.DMA((5,)),
            pltpu.SemaphoreType.DMA((5,)),
            pltpu.SemaphoreType.DMA((5,)),
            pltpu.SemaphoreType.DMA((5,)),
        ],
        compiler_params=pltpu.CompilerParams(collective_id=0),
    )(x2, Wq, k2, v2, Wo)
    return out.reshape(1, SQ, D)


# device time: 92923 ns/iter; 2.4978x vs baseline; 1.0985x over previous
import jax
import jax.numpy as jnp
from jax import lax
from jax.experimental import pallas as pl
from jax.experimental.pallas import tpu as pltpu

N_DEV = 32
SQ = 1024
HQ = 8
DH = 128
D = HQ * DH
SKV_LOCAL = 1024
R = 4
QG = SQ // R
KG = SKV_LOCAL // R
CHUNK = SQ // N_DEV
EXT = D + 128
SCALE = 0.08838834764831843

RS_M = (1, 8, 2, 4, 16)
RS_S = (16, 8, 4, 2, 1)
RS_OFF = (0, 512, 768, 896, 960)
AG_M = (16, 4, 2, 8, 1)


def _orig_start(c):
    r = c // 8
    a = (c % 8) // 2
    t0 = (c % 2) * 32
    return (4 * a + r) * 64 + t0


def kernel(x, Wq, K_ext, V_ext, Wo):
    x2 = x.reshape(SQ, D)
    k2 = K_ext.reshape(SKV_LOCAL, HQ, DH)
    v2 = V_ext.reshape(SKV_LOCAL, HQ, DH)

    def body(x_ref, wq_ref, k_ref, v_ref, wo_ref, out_ref,
             qbf_ref, qp_ref, kp_ref, vp_ref, wobf_ref,
             oext_ref, ctx_ref, send_buf, recv_buf, out_perm,
             rs_send_sems, rs_recv_sems, ag_send_sems, ag_recv_sems):
        my = lax.axis_index("i")

        barrier_sem = pltpu.get_barrier_semaphore()
        for m in RS_M:
            pl.semaphore_signal(
                barrier_sem, inc=1,
                device_id=(my ^ m,), device_id_type=pl.DeviceIdType.MESH,
            )
        pl.semaphore_wait(barrier_sem, len(RS_M))

        wobf_ref[...] = wo_ref[...].astype(jnp.bfloat16)
        qbf_ref[...] = jnp.dot(
            x_ref[...].astype(jnp.bfloat16),
            wq_ref[...].astype(jnp.bfloat16),
            preferred_element_type=jnp.float32,
        ).astype(jnp.bfloat16)
        for t in range(16):
            r, a = t % 4, t // 4
            dst = r * 256 + a * 64
            qp_ref[dst:dst + 64, :] = qbf_ref[t * 64:(t + 1) * 64, :]
            kp_ref[dst:dst + 64, :] = (
                k_ref[t * 64:(t + 1) * 64, :, :].reshape(64, D)
                .astype(jnp.bfloat16))
            vp_ref[dst:dst + 64, :] = (
                v_ref[t * 64:(t + 1) * 64, :, :].reshape(64, D)
                .astype(jnp.bfloat16))

        oext_ref[:, D:] = jnp.zeros((SQ, 128), jnp.float32)
        for r in range(R):
            rows = slice(r * QG, (r + 1) * QG)
            for h in range(HQ):
                cols = slice(h * DH, (h + 1) * DH)
                qh = qp_ref[rows, cols]
                kh = kp_ref[rows, cols]
                s = lax.dot_general(
                    qh, kh, (((1,), (1,)), ((), ())),
                    preferred_element_type=jnp.float32)
                w = jnp.exp(s * SCALE)
                o = jnp.dot(w.astype(jnp.bfloat16), vp_ref[rows, cols],
                            preferred_element_type=jnp.float32)
                oext_ref[rows, cols] = o
                oext_ref[rows, D + h:D + h + 1] = jnp.sum(
                    w, axis=1, keepdims=True)

        lo = my * 0
        for k in range(5):
            m, s_ch, off = RS_M[k], RS_S[k], RS_OFF[k]
            rows = s_ch * CHUNK
            bit = (my // m) % 2
            keep_lo = lo + bit * s_ch
            send_lo = lo + (1 - bit) * s_ch
            send_buf[0:rows, :] = (
                oext_ref[pl.ds(send_lo * CHUNK, rows), :]
                .astype(jnp.bfloat16))
            rdma = pltpu.make_async_remote_copy(
                src_ref=send_buf.at[pl.ds(0, rows)],
                dst_ref=recv_buf.at[pl.ds(off, rows)],
                send_sem=rs_send_sems.at[k],
                recv_sem=rs_recv_sems.at[k],
                device_id=(my ^ m,),
                device_id_type=pl.DeviceIdType.MESH,
            )
            rdma.start()
            rdma.wait()
            oext_ref[pl.ds(keep_lo * CHUNK, rows), :] = (
                oext_ref[pl.ds(keep_lo * CHUNK, rows), :]
                + recv_buf[off:off + rows, :].astype(jnp.float32))
            lo = keep_lo

        for h in range(HQ):
            oh = oext_ref[pl.ds(lo * CHUNK, CHUNK), h * DH:(h + 1) * DH]
            lh = oext_ref[pl.ds(lo * CHUNK, CHUNK), D + h:D + h + 1]
            ctx_ref[:, h * DH:(h + 1) * DH] = (oh / lh).astype(jnp.bfloat16)
        out_chunk = jnp.dot(ctx_ref[...], wobf_ref[...],
                            preferred_element_type=jnp.float32)
        out_perm[pl.ds(lo * CHUNK, CHUNK), :] = out_chunk.astype(jnp.bfloat16)

        def copy_out(c0, n):
            for i in range(n):
                c = c0 + i
                st = ((4 * ((c % 8) // 2) + c // 8) * 64
                      + (c % 2) * CHUNK)
                out_ref[pl.ds(st, CHUNK), :] = (
                    out_perm[pl.ds(c * CHUNK, CHUNK), :]
                    .astype(jnp.float32))

        glo, gs = lo, 1
        prev = (lo, 1)
        rdmas = []
        for j in range(5):
            rows = gs * CHUNK
            rdma = pltpu.make_async_remote_copy(
                src_ref=out_perm.at[pl.ds(glo * CHUNK, rows)],
                dst_ref=out_perm.at[pl.ds(glo * CHUNK, rows)],
                send_sem=ag_send_sems.at[j],
                recv_sem=ag_recv_sems.at[j],
                device_id=(my ^ AG_M[j],),
                device_id_type=pl.DeviceIdType.MESH,
            )
            rdma.start()
            rdmas.append(rdma)
            copy_out(*prev)
            rdma.wait_recv()
            bitj = (glo // gs) % 2
            prev = (glo + (1 - 2 * bitj) * gs, gs)
            glo = (glo // (2 * gs)) * (2 * gs)
            gs *= 2
        copy_out(*prev)
        for rdma in rdmas:
            rdma.wait_send()

    out = pl.pallas_call(
        body,
        out_shape=jax.ShapeDtypeStruct((SQ, D), jnp.float32),
        in_specs=[pl.BlockSpec(memory_space=pltpu.VMEM)] * 5,
        out_specs=pl.BlockSpec(memory_space=pltpu.VMEM),
        scratch_shapes=[
            pltpu.VMEM((SQ, D), jnp.bfloat16),
            pltpu.VMEM((SQ, D), jnp.bfloat16),
            pltpu.VMEM((SKV_LOCAL, D), jnp.bfloat16),
            pltpu.VMEM((SKV_LOCAL, D), jnp.bfloat16),
            pltpu.VMEM((D, D), jnp.bfloat16),
            pltpu.VMEM((SQ, EXT), jnp.float32),
            pltpu.VMEM((CHUNK, D), jnp.bfloat16),
            pltpu.VMEM((512, EXT), jnp.bfloat16),
            pltpu.VMEM((992, EXT), jnp.bfloat16),
            pltpu.VMEM((SQ, D), jnp.bfloat16),
            pltpu.SemaphoreType.DMA((5,)),
            pltpu.SemaphoreType.DMA((5,)),
            pltpu.SemaphoreType.DMA((5,)),
            pltpu.SemaphoreType.DMA((5,)),
        ],
        compiler_params=pltpu.CompilerParams(collective_id=0),
    )(x2, Wq, k2, v2, Wo)
    return out.reshape(1, SQ, D)
